# baseline (speedup 1.0000x reference)
import functools

import jax
import jax.numpy as jnp
from jax import lax
from jax.experimental import pallas as pl
from jax.experimental.pallas import tpu as pltpu
from jax.experimental.pallas import tpu_sc as plsc

N_NODES = 10000
DIM = 128
N_EDGES = 320000

NC = 2
NS = 16
NW = NC * NS
CHUNK = 64
E_PAD = 327680
R_TOT = E_PAD // CHUNK
N_ACC = 10112
ROWS_PER_TILE = N_ACC // NS
IDX_G = 16
NBUF = 4
NCH0 = 320
NCH1 = 0
NCH_D = R_TOT // NW
DEG_W = 128


def _feat_body(x_hbm, src_hbm, dst_hbm, zf_hbm,
               acc_out,
               acc_sh, src_v, dst_v, *bufs_and_sems):
    msgs = bufs_and_sems[:NBUF]
    gsems = bufs_and_sems[NBUF:2 * NBUF]
    ssems = bufs_and_sems[2 * NBUF:3 * NBUF]
    cid = lax.axis_index("c")
    sid = lax.axis_index("s")
    r0 = sid * ROWS_PER_TILE
    pltpu.sync_copy(zf_hbm, acc_sh.at[pl.ds(r0, ROWS_PER_TILE)])
    row0 = jnp.where(cid == 0, sid * NCH0, NS * NCH0 + sid * NCH1)
    ngroups = jnp.where(cid == 0, NCH0 // IDX_G, NCH1 // IDX_G)
    plsc.subcore_barrier()

    depth = NBUF - 1

    def group(g, carry):
        pltpu.sync_copy(src_hbm.at[pl.ds(row0 + g * IDX_G, IDX_G)], src_v)
        pltpu.sync_copy(dst_hbm.at[pl.ds(row0 + g * IDX_G, IDX_G)], dst_v)
        gd, sd = {}, {}
        for j in range(depth):
            gd[j] = pltpu.async_copy(x_hbm.at[src_v.at[j]],
                                     msgs[j % NBUF], gsems[j % NBUF])
        for j in range(IDX_G):
            cur = j % NBUF
            gd[j].wait()
            nj = j + depth
            if nj < IDX_G:
                if j >= 1:
                    sd[j - 1].wait()
                gd[nj] = pltpu.async_copy(x_hbm.at[src_v.at[nj]],
                                          msgs[nj % NBUF], gsems[nj % NBUF])
            sd[j] = pltpu.async_copy(msgs[cur], acc_sh.at[dst_v.at[j]],
                                     ssems[cur], add=True)
        for j in range(IDX_G - depth - 1, IDX_G):
            if j >= 0:
                sd[j].wait()
        return carry

    lax.fori_loop(0, ngroups, group, 0)
    plsc.subcore_barrier()
    pltpu.sync_copy(acc_sh.at[pl.ds(r0, ROWS_PER_TILE)],
                    acc_out.at[cid, pl.ds(r0, ROWS_PER_TILE)])


def _feat_agg(x, src2, dst2, zf):
    mesh = plsc.VectorSubcoreMesh(core_axis_name="c", subcore_axis_name="s")
    return pl.kernel(
        _feat_body,
        out_type=jax.ShapeDtypeStruct((NC, N_ACC, DIM), jnp.float32),
        mesh=mesh,
        scratch_types=[
            pltpu.VMEM_SHARED((N_ACC, DIM), jnp.float32),
            pltpu.VMEM((IDX_G, CHUNK), jnp.int32),
            pltpu.VMEM((IDX_G, CHUNK), jnp.int32),
        ] + [pltpu.VMEM((CHUNK, DIM), jnp.float32) for _ in range(NBUF)]
          + [pltpu.SemaphoreType.DMA for _ in range(2 * NBUF)],
    )(x, src2, dst2, zf)


def _deg_body(dst_hbm, ones_hbm, zf_hbm,
              deg_out,
              deg_sh, dst_v, ones_v, sem):
    cid = lax.axis_index("c")
    sid = lax.axis_index("s")
    wid = cid * NS + sid
    r0 = sid * ROWS_PER_TILE
    pltpu.sync_copy(zf_hbm, deg_sh.at[pl.ds(r0, ROWS_PER_TILE)])
    pltpu.sync_copy(ones_hbm, ones_v)
    row0 = wid * NCH_D
    plsc.subcore_barrier()

    def outer(g, carry):
        pltpu.sync_copy(dst_hbm.at[pl.ds(row0 + g * IDX_G, IDX_G)], dst_v)
        descs = [pltpu.async_copy(ones_v, deg_sh.at[dst_v.at[j]], sem,
                                  add=True)
                 for j in range(IDX_G)]
        for d in descs:
            d.wait()
        return carry

    lax.fori_loop(0, NCH_D // IDX_G, outer, 0)
    plsc.subcore_barrier()
    pltpu.sync_copy(deg_sh.at[pl.ds(r0, ROWS_PER_TILE)],
                    deg_out.at[cid, pl.ds(r0, ROWS_PER_TILE)])


def _deg_sc(dst2, ones, zf):
    mesh = plsc.VectorSubcoreMesh(core_axis_name="c", subcore_axis_name="s")
    return pl.kernel(
        _deg_body,
        out_type=jax.ShapeDtypeStruct((NC, N_ACC, DEG_W), jnp.float32),
        mesh=mesh,
        scratch_types=[
            pltpu.VMEM_SHARED((N_ACC, DEG_W), jnp.float32),
            pltpu.VMEM((IDX_G, CHUNK), jnp.int32),
            pltpu.VMEM((CHUNK, DEG_W), jnp.float32),
            pltpu.SemaphoreType.DMA,
        ],
    )(dst2, ones, zf)


def _tc_layer_body(x_ref, acc_ref, deg_ref, ws_ref, wn_ref, b_ref, o_ref,
                   *, relu):
    agg = acc_ref[0] + acc_ref[1]
    deg = deg_ref[0, :, 0:1] + deg_ref[1, :, 0:1]
    mean = agg * (1.0 / jnp.maximum(deg, 1.0))
    h = (jnp.dot(x_ref[...], ws_ref[...],
                 preferred_element_type=jnp.float32,
                 precision=lax.Precision.HIGHEST)
         + jnp.dot(mean, wn_ref[...],
                   preferred_element_type=jnp.float32,
                   precision=lax.Precision.HIGHEST)
         + b_ref[...])
    o_ref[...] = jnp.maximum(h, 0.0) if relu else h


def _tc_layer(x, acc, deg, w_self, w_neigh, b, relu):
    br = 2000
    grid = (N_NODES // br,)
    return pl.pallas_call(
        functools.partial(_tc_layer_body, relu=relu),
        grid=grid,
        in_specs=[
            pl.BlockSpec((br, DIM), lambda i: (i, 0)),
            pl.BlockSpec((NC, br, DIM), lambda i: (0, i, 0)),
            pl.BlockSpec((NC, br, DEG_W), lambda i: (0, i, 0)),
            pl.BlockSpec((DIM, DIM), lambda i: (0, 0)),
            pl.BlockSpec((DIM, DIM), lambda i: (0, 0)),
            pl.BlockSpec((1, DIM), lambda i: (0, 0)),
        ],
        out_specs=pl.BlockSpec((br, DIM), lambda i: (i, 0)),
        out_shape=jax.ShapeDtypeStruct((N_NODES, DIM), jnp.float32),
    )(x, acc, deg, w_self, w_neigh, b.reshape(1, DIM))


def kernel(x, edge_index, W1_self, W1_neigh, b1, W2_self, W2_neigh, b2):
    ei = edge_index.astype(jnp.int32)
    pad = E_PAD - N_EDGES
    src2 = jnp.concatenate(
        [ei[0], jnp.zeros((pad,), jnp.int32)]).reshape(R_TOT, CHUNK)
    dst_pad = N_NODES + (jnp.arange(pad, dtype=jnp.int32) % (N_ACC - N_NODES))
    dst2 = jnp.concatenate([ei[1], dst_pad]).reshape(R_TOT, CHUNK)
    zf = jnp.zeros((ROWS_PER_TILE, DIM), jnp.float32)
    ones = jnp.ones((CHUNK, DEG_W), jnp.float32)

    deg1 = _deg_sc(dst2, ones, zf)
    deg1, x_seq = lax.optimization_barrier((deg1, x))
    agg1 = _feat_agg(x_seq, src2, dst2, zf)
    h1 = _tc_layer(x, agg1, deg1, W1_self, W1_neigh, b1, relu=True)
    agg2 = _feat_agg(h1, src2, dst2, zf)
    out = _tc_layer(h1, agg2, deg1, W2_self, W2_neigh, b2, relu=False)
    return out


# speedup vs baseline: 1.4923x; 1.4923x over previous
import functools

import jax
import jax.numpy as jnp
from jax import lax
from jax.experimental import pallas as pl
from jax.experimental.pallas import tpu as pltpu
from jax.experimental.pallas import tpu_sc as plsc

N_NODES = 10000
DIM = 128
N_EDGES = 320000

NC = 2
NS = 16
NW = NC * NS
CHUNK = 64
E_PAD = 327680
R_TOT = E_PAD // CHUNK
N_ACC = 10112
ROWS_PER_TILE = N_ACC // NS
IDX_G = 16
NBUF = 4
NCH0 = 304
NCH1 = 16
NCH_D = R_TOT // NW
DEG_W = 128


def _feat_body(x_hbm, src_hbm, dst_hbm, zf_hbm,
               acc_out,
               acc_sh, src_v, dst_v, *bufs_and_sems):
    msgs = bufs_and_sems[:NBUF]
    gsems = bufs_and_sems[NBUF:2 * NBUF]
    ssems = bufs_and_sems[2 * NBUF:3 * NBUF]
    cid = lax.axis_index("c")
    sid = lax.axis_index("s")
    r0 = sid * ROWS_PER_TILE
    pltpu.sync_copy(zf_hbm, acc_sh.at[pl.ds(r0, ROWS_PER_TILE)])
    row0 = jnp.where(cid == 0, sid * NCH0, NS * NCH0 + sid * NCH1)
    ngroups = jnp.where(cid == 0, NCH0 // IDX_G, NCH1 // IDX_G)
    plsc.subcore_barrier()

    depth = NBUF - 1

    def group(g, carry):
        pltpu.sync_copy(src_hbm.at[pl.ds(row0 + g * IDX_G, IDX_G)], src_v)
        pltpu.sync_copy(dst_hbm.at[pl.ds(row0 + g * IDX_G, IDX_G)], dst_v)
        gd, sd = {}, {}
        for j in range(depth):
            gd[j] = pltpu.async_copy(x_hbm.at[src_v.at[j]],
                                     msgs[j % NBUF], gsems[j % NBUF])
        for j in range(IDX_G):
            cur = j % NBUF
            gd[j].wait()
            nj = j + depth
            if nj < IDX_G:
                if j >= 1:
                    sd[j - 1].wait()
                gd[nj] = pltpu.async_copy(x_hbm.at[src_v.at[nj]],
                                          msgs[nj % NBUF], gsems[nj % NBUF])
            sd[j] = pltpu.async_copy(msgs[cur], acc_sh.at[dst_v.at[j]],
                                     ssems[cur], add=True)
        for j in range(IDX_G - depth - 1, IDX_G):
            if j >= 0:
                sd[j].wait()
        return carry

    lax.fori_loop(0, ngroups, group, 0)
    plsc.subcore_barrier()
    pltpu.sync_copy(acc_sh.at[pl.ds(r0, ROWS_PER_TILE)],
                    acc_out.at[cid, pl.ds(r0, ROWS_PER_TILE)])


def _feat_agg(x, src2, dst2, zf):
    mesh = plsc.VectorSubcoreMesh(core_axis_name="c", subcore_axis_name="s")
    return pl.kernel(
        _feat_body,
        out_type=jax.ShapeDtypeStruct((NC, N_ACC, DIM), jnp.float32),
        mesh=mesh,
        scratch_types=[
            pltpu.VMEM_SHARED((N_ACC, DIM), jnp.float32),
            pltpu.VMEM((IDX_G, CHUNK), jnp.int32),
            pltpu.VMEM((IDX_G, CHUNK), jnp.int32),
        ] + [pltpu.VMEM((CHUNK, DIM), jnp.float32) for _ in range(NBUF)]
          + [pltpu.SemaphoreType.DMA for _ in range(2 * NBUF)],
    )(x, src2, dst2, zf)


def _deg_body(dst_hbm, ones_hbm, zf_hbm,
              deg_out,
              deg_sh, dst_v, ones_v, sem):
    cid = lax.axis_index("c")
    sid = lax.axis_index("s")
    wid = cid * NS + sid
    r0 = sid * ROWS_PER_TILE
    pltpu.sync_copy(zf_hbm, deg_sh.at[pl.ds(r0, ROWS_PER_TILE)])
    pltpu.sync_copy(ones_hbm, ones_v)
    row0 = wid * NCH_D
    plsc.subcore_barrier()

    def outer(g, carry):
        pltpu.sync_copy(dst_hbm.at[pl.ds(row0 + g * IDX_G, IDX_G)], dst_v)
        descs = [pltpu.async_copy(ones_v, deg_sh.at[dst_v.at[j]], sem,
                                  add=True)
                 for j in range(IDX_G)]
        for d in descs:
            d.wait()
        return carry

    lax.fori_loop(0, NCH_D // IDX_G, outer, 0)
    plsc.subcore_barrier()
    pltpu.sync_copy(deg_sh.at[pl.ds(r0, ROWS_PER_TILE)],
                    deg_out.at[cid, pl.ds(r0, ROWS_PER_TILE)])


def _deg_sc(dst2, ones, zf):
    mesh = plsc.VectorSubcoreMesh(core_axis_name="c", subcore_axis_name="s")
    return pl.kernel(
        _deg_body,
        out_type=jax.ShapeDtypeStruct((NC, N_ACC, DEG_W), jnp.float32),
        mesh=mesh,
        scratch_types=[
            pltpu.VMEM_SHARED((N_ACC, DEG_W), jnp.float32),
            pltpu.VMEM((IDX_G, CHUNK), jnp.int32),
            pltpu.VMEM((CHUNK, DEG_W), jnp.float32),
            pltpu.SemaphoreType.DMA,
        ],
    )(dst2, ones, zf)


def _tc_layer_body(x_ref, acc_ref, deg_ref, ws_ref, wn_ref, b_ref, o_ref,
                   *, relu):
    agg = acc_ref[0] + acc_ref[1]
    deg = deg_ref[0, :, 0:1] + deg_ref[1, :, 0:1]
    mean = agg * (1.0 / jnp.maximum(deg, 1.0))
    h = (jnp.dot(x_ref[...], ws_ref[...],
                 preferred_element_type=jnp.float32,
                 precision=lax.Precision.HIGHEST)
         + jnp.dot(mean, wn_ref[...],
                   preferred_element_type=jnp.float32,
                   precision=lax.Precision.HIGHEST)
         + b_ref[...])
    o_ref[...] = jnp.maximum(h, 0.0) if relu else h


def _tc_layer(x, acc, deg, w_self, w_neigh, b, relu):
    br = 2000
    grid = (N_NODES // br,)
    return pl.pallas_call(
        functools.partial(_tc_layer_body, relu=relu),
        grid=grid,
        in_specs=[
            pl.BlockSpec((br, DIM), lambda i: (i, 0)),
            pl.BlockSpec((NC, br, DIM), lambda i: (0, i, 0)),
            pl.BlockSpec((NC, br, DEG_W), lambda i: (0, i, 0)),
            pl.BlockSpec((DIM, DIM), lambda i: (0, 0)),
            pl.BlockSpec((DIM, DIM), lambda i: (0, 0)),
            pl.BlockSpec((1, DIM), lambda i: (0, 0)),
        ],
        out_specs=pl.BlockSpec((br, DIM), lambda i: (i, 0)),
        out_shape=jax.ShapeDtypeStruct((N_NODES, DIM), jnp.float32),
    )(x, acc, deg, w_self, w_neigh, b.reshape(1, DIM))


def kernel(x, edge_index, W1_self, W1_neigh, b1, W2_self, W2_neigh, b2):
    ei = edge_index.astype(jnp.int32)
    pad = E_PAD - N_EDGES
    src2 = jnp.concatenate(
        [ei[0], jnp.zeros((pad,), jnp.int32)]).reshape(R_TOT, CHUNK)
    dst_pad = N_NODES + (jnp.arange(pad, dtype=jnp.int32) % (N_ACC - N_NODES))
    dst2 = jnp.concatenate([ei[1], dst_pad]).reshape(R_TOT, CHUNK)
    zf = jnp.zeros((ROWS_PER_TILE, DIM), jnp.float32)
    ones = jnp.ones((CHUNK, DEG_W), jnp.float32)

    deg1 = _deg_sc(dst2, ones, zf)
    deg1, x_seq = lax.optimization_barrier((deg1, x))
    agg1 = _feat_agg(x_seq, src2, dst2, zf)
    h1 = _tc_layer(x, agg1, deg1, W1_self, W1_neigh, b1, relu=True)
    agg2 = _feat_agg(h1, src2, dst2, zf)
    out = _tc_layer(h1, agg2, deg1, W2_self, W2_neigh, b2, relu=False)
    return out
